# baseline (speedup 1.0000x reference)
import functools

import jax
import jax.numpy as jnp
from jax import lax
from jax.experimental import pallas as pl
from jax.experimental.pallas import tpu as pltpu
from jax.experimental.pallas import tpu_sc as plsc

N = 10000
E = 320000
D = 128

NC = 2
NS = 16
NW = NC * NS
EPW = E // NW
C = 80
NCHUNK = EPW // C
TOTCH = E // C
GPC = C // 16
NPAD = 10240
NPT = NPAD // NS

R = 400
GRID = N // R
RP = 512
GRIDP = NPAD // RP


def _sc_edge_body(xl_hbm, xr_hbm, att_hbm, src_hbm, dst_hbm, znd_hbm, zn_hbm,
                  num_out, den_out,
                  num_sh, den_sh, attv,
                  srcv0, dstv0, dstsc0, xlbuf0, xrbuf0, exbuf0,
                  srcv1, dstv1, dstsc1, xlbuf1, xrbuf1, exbuf1,
                  accbuf, semg0, semg1, semsc0, semsc1, semi0, semi1):
    cid = lax.axis_index("c")
    sid = lax.axis_index("s")
    wid = sid * NC + cid
    bufs = [(srcv0, dstv0, dstsc0, xlbuf0, xrbuf0, exbuf0, semg0, semsc0, semi0),
            (srcv1, dstv1, dstsc1, xlbuf1, xrbuf1, exbuf1, semg1, semsc1, semi1)]

    pltpu.sync_copy(znd_hbm.at[pl.ds(sid * NPT, NPT)],
                    num_sh.at[pl.ds(sid * NPT, NPT)])

    @pl.when(sid == 0)
    def _():
        pltpu.sync_copy(zn_hbm, den_sh)

    pltpu.sync_copy(att_hbm, attv)
    plsc.subcore_barrier()

    att_regs = [attv[pl.ds(16 * j, 16)] for j in range(8)]
    row_iota = lax.iota(jnp.int32, 16)

    def fire_first(i, b):
        srcv, dstv, _, xlbuf, xrbuf, _, semg, _, _ = bufs[b]
        base = (wid * NCHUNK + i) * C
        pltpu.sync_copy(src_hbm.at[pl.ds(base, C)], srcv)
        pltpu.sync_copy(dst_hbm.at[pl.ds(base, C)], dstv)
        pltpu.async_copy(xl_hbm.at[srcv], xlbuf, semg)
        pltpu.async_copy(xr_hbm.at[dstv], xrbuf, semg)

    def fire(i, b):
        srcv, dstv, _, xlbuf, xrbuf, _, semg, _, semi = bufs[b]
        base = (wid * NCHUNK + i) * C
        pltpu.make_async_copy(src_hbm.at[pl.ds(base, C)], srcv, semi).wait()
        pltpu.make_async_copy(dst_hbm.at[pl.ds(base, C)], dstv, semi).wait()
        pltpu.async_copy(xl_hbm.at[srcv], xlbuf, semg)
        pltpu.async_copy(xr_hbm.at[dstv], xrbuf, semg)

    def drain_scatters(b):
        _, _, dstsc, xlbuf, _, exbuf, _, semsc, _ = bufs[b]
        pltpu.make_async_copy(xlbuf, num_sh.at[dstsc], semsc).wait()
        pltpu.make_async_copy(exbuf, den_sh.at[dstsc], semsc).wait()

    def start(i, b):
        drain_scatters(b)
        fire(i, b)

    def compute(i, b, prefetch=True):
        srcv, dstv, dstsc, xlbuf, xrbuf, exbuf, semg, semsc, semi = bufs[b]
        pltpu.make_async_copy(xl_hbm.at[srcv], xlbuf, semg).wait()
        pltpu.make_async_copy(xr_hbm.at[dstv], xrbuf, semg).wait()
        for g in range(GPC):
            dstsc[pl.ds(g * 16, 16)] = dstv[pl.ds(g * 16, 16)]
        if prefetch:
            base2 = (wid * NCHUNK + i + 2) * C
            pltpu.async_copy(src_hbm.at[pl.ds(base2, C)], srcv, semi)
            pltpu.async_copy(dst_hbm.at[pl.ds(base2, C)], dstv, semi)
        for g in range(C // 16):
            @plsc.parallel_loop(0, 16, unroll=4)
            def _(k):
                e = g * 16 + k
                acc = jnp.zeros((16,), jnp.float32)
                for j in range(8):
                    v = xlbuf[e, pl.ds(16 * j, 16)] + xrbuf[e, pl.ds(16 * j, 16)]
                    v = v * jnp.where(v > 0.0, 1.0, 0.2)
                    acc = acc + v * att_regs[j]
                accbuf[pl.ds(k * 16, 16)] = acc
            logits = plsc.load_gather(accbuf, [row_iota * 16])
            for k in range(1, 16):
                logits = logits + plsc.load_gather(accbuf, [row_iota * 16 + k])
            exbuf[pl.ds(g * 16, 16)] = jnp.exp(logits)

            @plsc.parallel_loop(0, 16, unroll=4)
            def _(k):
                e = g * 16 + k
                exs = plsc.load_gather(exbuf, [jnp.full((16,), e, jnp.int32)])
                for j in range(8):
                    xlbuf[e, pl.ds(16 * j, 16)] = (
                        xlbuf[e, pl.ds(16 * j, 16)] * exs)

        pltpu.async_copy(xlbuf, num_sh.at[dstsc], semsc, add=True)
        pltpu.async_copy(exbuf, den_sh.at[dstsc], semsc, add=True)

    fire_first(0, 0)
    fire_first(1, 1)
    compute(0, 0)

    def pair_body(p, carry):
        c0 = 2 * p + 2
        start(c0, 0)
        compute(c0 - 1, 1)
        start(c0 + 1, 1)
        compute(c0, 0)
        return carry

    lax.fori_loop(0, (NCHUNK - 3) // 2, pair_body, 0)
    start(NCHUNK - 1, 0)
    compute(NCHUNK - 2, 1, prefetch=False)
    compute(NCHUNK - 1, 0, prefetch=False)
    drain_scatters(1)
    drain_scatters(0)
    plsc.subcore_barrier()

    pltpu.sync_copy(num_sh.at[pl.ds(sid * NPT, NPT)],
                    num_out.at[cid, pl.ds(sid * NPT, NPT)])

    @pl.when(sid == 0)
    def _():
        pltpu.sync_copy(den_sh, den_out.at[cid, 0])


_sc_edge = pl.kernel(
    _sc_edge_body,
    out_type=(
        jax.ShapeDtypeStruct((NC, NPAD, D), jnp.float32),
        jax.ShapeDtypeStruct((NC, 1, NPAD), jnp.float32),
    ),
    mesh=plsc.VectorSubcoreMesh(core_axis_name="c", subcore_axis_name="s"),
    compiler_params=pltpu.CompilerParams(needs_layout_passes=False),
    scratch_types=[
        pltpu.VMEM_SHARED((NPAD, D), jnp.float32),
        pltpu.VMEM_SHARED((NPAD,), jnp.float32),
        pltpu.VMEM((D,), jnp.float32),
        pltpu.VMEM((C,), jnp.int32),
        pltpu.VMEM((C,), jnp.int32),
        pltpu.VMEM((C,), jnp.int32),
        pltpu.VMEM((C, D), jnp.float32),
        pltpu.VMEM((C, D), jnp.float32),
        pltpu.VMEM((C,), jnp.float32),
        pltpu.VMEM((C,), jnp.int32),
        pltpu.VMEM((C,), jnp.int32),
        pltpu.VMEM((C,), jnp.int32),
        pltpu.VMEM((C, D), jnp.float32),
        pltpu.VMEM((C, D), jnp.float32),
        pltpu.VMEM((C,), jnp.float32),
        pltpu.VMEM((256,), jnp.float32),
        pltpu.SemaphoreType.DMA,
        pltpu.SemaphoreType.DMA,
        pltpu.SemaphoreType.DMA,
        pltpu.SemaphoreType.DMA,
        pltpu.SemaphoreType.DMA,
        pltpu.SemaphoreType.DMA,
    ],
)


def _proj_body(x_ref, wl_ref, wr_ref, xl_ref, xr_ref):
    h = x_ref[...]
    xl_ref[...] = jnp.dot(h, wl_ref[...], preferred_element_type=jnp.float32)
    xr_ref[...] = jnp.dot(h, wr_ref[...], preferred_element_type=jnp.float32)


def _tc_proj(x, wl, wr):
    return pl.pallas_call(
        _proj_body,
        grid=(GRID,),
        in_specs=[
            pl.BlockSpec((R, D), lambda i: (i, 0)),
            pl.BlockSpec((D, D), lambda i: (0, 0)),
            pl.BlockSpec((D, D), lambda i: (0, 0)),
        ],
        out_specs=[
            pl.BlockSpec((R, D), lambda i: (i, 0)),
            pl.BlockSpec((R, D), lambda i: (i, 0)),
        ],
        out_shape=[
            jax.ShapeDtypeStruct((N, D), jnp.float32),
            jax.ShapeDtypeStruct((N, D), jnp.float32),
        ],
    )(x, wl, wr)


def _combine_h(num_ref, den_ref):
    i = pl.program_id(0)
    den = den_ref[0, 0, pl.ds(i * RP, RP)] + den_ref[1, 0, pl.ds(i * RP, RP)]
    num = num_ref[0] + num_ref[1]
    den = den[:, None]
    h = jnp.where(den > 0.0, num / den, 0.0)
    return jnp.where(h > 0.0, h, jnp.exp(h) - 1.0)


def _mid_body(num_ref, den_ref, wl_ref, wr_ref, xl_ref, xr_ref):
    h = _combine_h(num_ref, den_ref)
    xl_ref[...] = jnp.dot(h, wl_ref[...], preferred_element_type=jnp.float32)
    xr_ref[...] = jnp.dot(h, wr_ref[...], preferred_element_type=jnp.float32)


def _tc_mid(num, den, wl, wr):
    return pl.pallas_call(
        _mid_body,
        grid=(GRIDP,),
        in_specs=[
            pl.BlockSpec((NC, RP, D), lambda i: (0, i, 0)),
            pl.BlockSpec((NC, 1, NPAD), lambda i: (0, 0, 0)),
            pl.BlockSpec((D, D), lambda i: (0, 0)),
            pl.BlockSpec((D, D), lambda i: (0, 0)),
        ],
        out_specs=[
            pl.BlockSpec((RP, D), lambda i: (i, 0)),
            pl.BlockSpec((RP, D), lambda i: (i, 0)),
        ],
        out_shape=[
            jax.ShapeDtypeStruct((NPAD, D), jnp.float32),
            jax.ShapeDtypeStruct((NPAD, D), jnp.float32),
        ],
    )(num, den, wl, wr)


def _final_body(num_ref, den_ref, out_ref):
    i = pl.program_id(0)
    den = den_ref[0, 0, pl.ds(i * RP, RP)] + den_ref[1, 0, pl.ds(i * RP, RP)]
    num = num_ref[0] + num_ref[1]
    den = den[:, None]
    h = jnp.where(den > 0.0, num / den, 0.0)
    out_ref[...] = jnp.where(h > 0.0, h, 128.0 * (jnp.exp(h) - 1.0))


def _tc_final(num, den):
    return pl.pallas_call(
        _final_body,
        grid=(GRIDP,),
        in_specs=[
            pl.BlockSpec((NC, RP, D), lambda i: (0, i, 0)),
            pl.BlockSpec((NC, 1, NPAD), lambda i: (0, 0, 0)),
        ],
        out_specs=pl.BlockSpec((RP, D), lambda i: (i, 0)),
        out_shape=jax.ShapeDtypeStruct((NPAD, D), jnp.float32),
    )(num, den)


def kernel(x, edge_index, Wl1, Wr1, att1, Wl2, Wr2, att2, Wl3, Wr3, att3,
           Wl4, Wr4, att4):
    src = edge_index[0]
    dst = edge_index[1]
    znd = jnp.zeros((NPAD, D), jnp.float32)
    zn = jnp.zeros((NPAD,), jnp.float32)

    xl, xr = _tc_proj(x, Wl1, Wr1)
    num, den = _sc_edge(xl, xr, att1, src, dst, znd, zn)
    xl, xr = _tc_mid(num, den, Wl2, Wr2)
    num, den = _sc_edge(xl, xr, att2, src, dst, znd, zn)
    xl, xr = _tc_mid(num, den, Wl3, Wr3)
    num, den = _sc_edge(xl, xr, att3, src, dst, znd, zn)
    xl, xr = _tc_mid(num, den, Wl4, Wr4)
    num, den = _sc_edge(xl, xr, att4, src, dst, znd, zn)
    return _tc_final(num, den)[:N]


# speedup vs baseline: 1.0302x; 1.0302x over previous
import functools

import jax
import jax.numpy as jnp
from jax import lax
from jax.experimental import pallas as pl
from jax.experimental.pallas import tpu as pltpu
from jax.experimental.pallas import tpu_sc as plsc

N = 10000
E = 320000
D = 128

NC = 2
NS = 16
NW = NC * NS
EPW = E // NW
C = 80
NCHUNK = EPW // C
TOTCH = E // C
GPC = C // 16
NPAD = 10240
NPT = NPAD // NS

R = 400
GRID = N // R
RP = 512
GRIDP = NPAD // RP


def _sc_edge_body(xl_hbm, xr_hbm, att_hbm, src_hbm, dst_hbm, znd_hbm, zn_hbm,
                  num_out, den_out,
                  num_sh, den_sh, attv,
                  srcv0, dstv0, dstg0, xlbuf0, xrbuf0, exbuf0,
                  srcv1, dstv1, dstg1, xlbuf1, xrbuf1, exbuf1,
                  accbuf, semg0, semg1, semsc0, semsc1, semi0, semi1):
    cid = lax.axis_index("c")
    sid = lax.axis_index("s")
    wid = sid * NC + cid
    bufs = [(srcv0, dstv0, dstg0, xlbuf0, xrbuf0, exbuf0, semg0, semsc0, semi0),
            (srcv1, dstv1, dstg1, xlbuf1, xrbuf1, exbuf1, semg1, semsc1, semi1)]

    pltpu.sync_copy(znd_hbm.at[pl.ds(sid * NPT, NPT)],
                    num_sh.at[pl.ds(sid * NPT, NPT)])

    @pl.when(sid == 0)
    def _():
        pltpu.sync_copy(zn_hbm, den_sh)

    pltpu.sync_copy(att_hbm, attv)
    plsc.subcore_barrier()

    att_regs = [attv[pl.ds(16 * j, 16)] for j in range(8)]
    row_iota = lax.iota(jnp.int32, 16)

    def fire_first(i, b):
        srcv, dstv, _, xlbuf, xrbuf, _, semg, _, _ = bufs[b]
        base = (wid * NCHUNK + i) * C
        pltpu.sync_copy(src_hbm.at[pl.ds(base, C)], srcv)
        pltpu.sync_copy(dst_hbm.at[pl.ds(base, C)], dstv)
        pltpu.async_copy(xl_hbm.at[srcv], xlbuf, semg)
        pltpu.async_copy(xr_hbm.at[dstv], xrbuf, semg)

    def fire(i, b):
        srcv, dstv, _, xlbuf, xrbuf, _, semg, _, semi = bufs[b]
        base = (wid * NCHUNK + i) * C
        pltpu.make_async_copy(src_hbm.at[pl.ds(base, C)], srcv, semi).wait()
        pltpu.make_async_copy(dst_hbm.at[pl.ds(base, C)], dstv, semi).wait()
        pltpu.async_copy(xl_hbm.at[srcv], xlbuf, semg)
        pltpu.async_copy(xr_hbm.at[dstv], xrbuf, semg)

    def drain_scatters(b):
        _, _, dstg, xlbuf, _, exbuf, _, semsc, _ = bufs[b]
        for g in range(GPC):
            pltpu.make_async_copy(
                xlbuf.at[pl.ds(g * 16, 16)],
                num_sh.at[dstg[g]], semsc).wait()
            pltpu.make_async_copy(
                exbuf.at[pl.ds(g * 16, 16)],
                den_sh.at[dstg[g]], semsc).wait()

    def start(i, b):
        drain_scatters(b)
        fire(i, b)

    def compute(i, b, prefetch=True):
        srcv, dstv, dstg, xlbuf, xrbuf, exbuf, semg, semsc, semi = bufs[b]
        pltpu.make_async_copy(xl_hbm.at[srcv], xlbuf, semg).wait()
        pltpu.make_async_copy(xr_hbm.at[dstv], xrbuf, semg).wait()
        for g in range(GPC):
            dstg[g][...] = dstv[pl.ds(g * 16, 16)]
        if prefetch:
            base2 = (wid * NCHUNK + i + 2) * C
            pltpu.async_copy(src_hbm.at[pl.ds(base2, C)], srcv, semi)
            pltpu.async_copy(dst_hbm.at[pl.ds(base2, C)], dstv, semi)
        for g in range(GPC + 1):
            @plsc.parallel_loop(0, 16, unroll=4)
            def _(k):
                if g < GPC:
                    e = g * 16 + k
                    acc = jnp.zeros((16,), jnp.float32)
                    for j in range(8):
                        v = (xlbuf[e, pl.ds(16 * j, 16)]
                             + xrbuf[e, pl.ds(16 * j, 16)])
                        v = v * jnp.where(v > 0.0, 1.0, 0.2)
                        acc = acc + v * att_regs[j]
                    accbuf[pl.ds(k * 16, 16)] = acc
                if g > 0:
                    e2 = (g - 1) * 16 + k
                    exs = plsc.load_gather(
                        exbuf, [jnp.full((16,), e2, jnp.int32)])
                    for j in range(8):
                        xlbuf[e2, pl.ds(16 * j, 16)] = (
                            xlbuf[e2, pl.ds(16 * j, 16)] * exs)
            if g < GPC:
                logits = plsc.load_gather(accbuf, [row_iota * 16])
                for k in range(1, 16):
                    logits = logits + plsc.load_gather(
                        accbuf, [row_iota * 16 + k])
                exbuf[pl.ds(g * 16, 16)] = jnp.exp(logits)
            if g > 0:
                pltpu.async_copy(xlbuf.at[pl.ds((g - 1) * 16, 16)],
                                 num_sh.at[dstg[g - 1]], semsc, add=True)
                pltpu.async_copy(exbuf.at[pl.ds((g - 1) * 16, 16)],
                                 den_sh.at[dstg[g - 1]], semsc, add=True)

    fire_first(0, 0)
    fire_first(1, 1)
    compute(0, 0)

    def pair_body(p, carry):
        c0 = 2 * p + 2
        start(c0, 0)
        compute(c0 - 1, 1)
        start(c0 + 1, 1)
        compute(c0, 0)
        return carry

    lax.fori_loop(0, (NCHUNK - 3) // 2, pair_body, 0)
    start(NCHUNK - 1, 0)
    compute(NCHUNK - 2, 1, prefetch=False)
    compute(NCHUNK - 1, 0, prefetch=False)
    drain_scatters(1)
    drain_scatters(0)
    plsc.subcore_barrier()

    pltpu.sync_copy(num_sh.at[pl.ds(sid * NPT, NPT)],
                    num_out.at[cid, pl.ds(sid * NPT, NPT)])

    @pl.when(sid == 0)
    def _():
        pltpu.sync_copy(den_sh, den_out.at[cid, 0])


_sc_edge = pl.kernel(
    _sc_edge_body,
    out_type=(
        jax.ShapeDtypeStruct((NC, NPAD, D), jnp.float32),
        jax.ShapeDtypeStruct((NC, 1, NPAD), jnp.float32),
    ),
    mesh=plsc.VectorSubcoreMesh(core_axis_name="c", subcore_axis_name="s"),
    compiler_params=pltpu.CompilerParams(needs_layout_passes=False),
    scratch_types=[
        pltpu.VMEM_SHARED((NPAD, D), jnp.float32),
        pltpu.VMEM_SHARED((NPAD,), jnp.float32),
        pltpu.VMEM((D,), jnp.float32),
        pltpu.VMEM((C,), jnp.int32),
        pltpu.VMEM((C,), jnp.int32),
        tuple(pltpu.VMEM((16,), jnp.int32) for _ in range(GPC)),
        pltpu.VMEM((C, D), jnp.float32),
        pltpu.VMEM((C, D), jnp.float32),
        pltpu.VMEM((C,), jnp.float32),
        pltpu.VMEM((C,), jnp.int32),
        pltpu.VMEM((C,), jnp.int32),
        tuple(pltpu.VMEM((16,), jnp.int32) for _ in range(GPC)),
        pltpu.VMEM((C, D), jnp.float32),
        pltpu.VMEM((C, D), jnp.float32),
        pltpu.VMEM((C,), jnp.float32),
        pltpu.VMEM((256,), jnp.float32),
        pltpu.SemaphoreType.DMA,
        pltpu.SemaphoreType.DMA,
        pltpu.SemaphoreType.DMA,
        pltpu.SemaphoreType.DMA,
        pltpu.SemaphoreType.DMA,
        pltpu.SemaphoreType.DMA,
    ],
)


def _proj_body(x_ref, wl_ref, wr_ref, xl_ref, xr_ref):
    h = x_ref[...]
    xl_ref[...] = jnp.dot(h, wl_ref[...], preferred_element_type=jnp.float32)
    xr_ref[...] = jnp.dot(h, wr_ref[...], preferred_element_type=jnp.float32)


def _tc_proj(x, wl, wr):
    return pl.pallas_call(
        _proj_body,
        grid=(GRID,),
        in_specs=[
            pl.BlockSpec((R, D), lambda i: (i, 0)),
            pl.BlockSpec((D, D), lambda i: (0, 0)),
            pl.BlockSpec((D, D), lambda i: (0, 0)),
        ],
        out_specs=[
            pl.BlockSpec((R, D), lambda i: (i, 0)),
            pl.BlockSpec((R, D), lambda i: (i, 0)),
        ],
        out_shape=[
            jax.ShapeDtypeStruct((N, D), jnp.float32),
            jax.ShapeDtypeStruct((N, D), jnp.float32),
        ],
    )(x, wl, wr)


def _combine_h(num_ref, den_ref):
    i = pl.program_id(0)
    den = den_ref[0, 0, pl.ds(i * RP, RP)] + den_ref[1, 0, pl.ds(i * RP, RP)]
    num = num_ref[0] + num_ref[1]
    den = den[:, None]
    h = jnp.where(den > 0.0, num / den, 0.0)
    return jnp.where(h > 0.0, h, jnp.exp(h) - 1.0)


def _mid_body(num_ref, den_ref, wl_ref, wr_ref, xl_ref, xr_ref):
    h = _combine_h(num_ref, den_ref)
    xl_ref[...] = jnp.dot(h, wl_ref[...], preferred_element_type=jnp.float32)
    xr_ref[...] = jnp.dot(h, wr_ref[...], preferred_element_type=jnp.float32)


def _tc_mid(num, den, wl, wr):
    return pl.pallas_call(
        _mid_body,
        grid=(GRIDP,),
        in_specs=[
            pl.BlockSpec((NC, RP, D), lambda i: (0, i, 0)),
            pl.BlockSpec((NC, 1, NPAD), lambda i: (0, 0, 0)),
            pl.BlockSpec((D, D), lambda i: (0, 0)),
            pl.BlockSpec((D, D), lambda i: (0, 0)),
        ],
        out_specs=[
            pl.BlockSpec((RP, D), lambda i: (i, 0)),
            pl.BlockSpec((RP, D), lambda i: (i, 0)),
        ],
        out_shape=[
            jax.ShapeDtypeStruct((NPAD, D), jnp.float32),
            jax.ShapeDtypeStruct((NPAD, D), jnp.float32),
        ],
    )(num, den, wl, wr)


def _final_body(num_ref, den_ref, out_ref):
    i = pl.program_id(0)
    den = den_ref[0, 0, pl.ds(i * RP, RP)] + den_ref[1, 0, pl.ds(i * RP, RP)]
    num = num_ref[0] + num_ref[1]
    den = den[:, None]
    h = jnp.where(den > 0.0, num / den, 0.0)
    out_ref[...] = jnp.where(h > 0.0, h, 128.0 * (jnp.exp(h) - 1.0))


def _tc_final(num, den):
    return pl.pallas_call(
        _final_body,
        grid=(GRIDP,),
        in_specs=[
            pl.BlockSpec((NC, RP, D), lambda i: (0, i, 0)),
            pl.BlockSpec((NC, 1, NPAD), lambda i: (0, 0, 0)),
        ],
        out_specs=pl.BlockSpec((RP, D), lambda i: (i, 0)),
        out_shape=jax.ShapeDtypeStruct((NPAD, D), jnp.float32),
    )(num, den)


def kernel(x, edge_index, Wl1, Wr1, att1, Wl2, Wr2, att2, Wl3, Wr3, att3,
           Wl4, Wr4, att4):
    src = edge_index[0]
    dst = edge_index[1]
    znd = jnp.zeros((NPAD, D), jnp.float32)
    zn = jnp.zeros((NPAD,), jnp.float32)

    xl, xr = _tc_proj(x, Wl1, Wr1)
    num, den = _sc_edge(xl, xr, att1, src, dst, znd, zn)
    xl, xr = _tc_mid(num, den, Wl2, Wr2)
    num, den = _sc_edge(xl, xr, att2, src, dst, znd, zn)
    xl, xr = _tc_mid(num, den, Wl3, Wr3)
    num, den = _sc_edge(xl, xr, att3, src, dst, znd, zn)
    xl, xr = _tc_mid(num, den, Wl4, Wr4)
    num, den = _sc_edge(xl, xr, att4, src, dst, znd, zn)
    return _tc_final(num, den)[:N]


# speedup vs baseline: 1.0630x; 1.0318x over previous
import functools

import jax
import jax.numpy as jnp
from jax import lax
from jax.experimental import pallas as pl
from jax.experimental.pallas import tpu as pltpu
from jax.experimental.pallas import tpu_sc as plsc

N = 10000
E = 320000
D = 128

NC = 2
NS = 16
NW = NC * NS
EPW = E // NW
C = 80
NCHUNK = EPW // C
TOTCH = E // C
GPC = C // 16
NPAD = 10240
NPT = NPAD // NS

R = 400
GRID = N // R
RP = 512
GRIDP = NPAD // RP


def _sc_edge_body(xl_hbm, xr_hbm, att_hbm, src_hbm, dst_hbm, znd_hbm, zn_hbm,
                  num_out, den_out,
                  num_sh, den_sh, attv,
                  srcv0, dstv0, dstg0, xlbuf0, xrbuf0, exbuf0,
                  srcv1, dstv1, dstg1, xlbuf1, xrbuf1, exbuf1,
                  accbuf, semg0, semg1, semsc0, semsc1, semi0, semi1):
    cid = lax.axis_index("c")
    sid = lax.axis_index("s")
    wid = sid * NC + cid
    bufs = [(srcv0, dstv0, dstg0, xlbuf0, xrbuf0, exbuf0, semg0, semsc0, semi0),
            (srcv1, dstv1, dstg1, xlbuf1, xrbuf1, exbuf1, semg1, semsc1, semi1)]

    pltpu.sync_copy(znd_hbm.at[pl.ds(sid * NPT, NPT)],
                    num_sh.at[pl.ds(sid * NPT, NPT)])

    @pl.when(sid == 0)
    def _():
        pltpu.sync_copy(zn_hbm, den_sh)

    pltpu.sync_copy(att_hbm, attv)
    plsc.subcore_barrier()

    att_regs = [attv[pl.ds(16 * j, 16)] for j in range(8)]
    row_iota = lax.iota(jnp.int32, 16)

    def fire_first(i, b):
        srcv, dstv, _, xlbuf, xrbuf, _, semg, _, _ = bufs[b]
        base = (wid * NCHUNK + i) * C
        pltpu.sync_copy(src_hbm.at[pl.ds(base, C)], srcv)
        pltpu.sync_copy(dst_hbm.at[pl.ds(base, C)], dstv)
        pltpu.async_copy(xl_hbm.at[srcv], xlbuf, semg)
        pltpu.async_copy(xr_hbm.at[dstv], xrbuf, semg)

    def fire(i, b):
        srcv, dstv, _, xlbuf, xrbuf, _, semg, _, semi = bufs[b]
        base = (wid * NCHUNK + i) * C
        pltpu.make_async_copy(src_hbm.at[pl.ds(base, C)], srcv, semi).wait()
        pltpu.make_async_copy(dst_hbm.at[pl.ds(base, C)], dstv, semi).wait()
        pltpu.async_copy(xl_hbm.at[srcv], xlbuf, semg)
        pltpu.async_copy(xr_hbm.at[dstv], xrbuf, semg)

    def drain_scatters(b):
        _, _, dstg, xlbuf, _, exbuf, _, semsc, _ = bufs[b]
        for g in range(GPC):
            pltpu.make_async_copy(
                xlbuf.at[pl.ds(g * 16, 16)],
                num_sh.at[dstg[g]], semsc).wait()
            pltpu.make_async_copy(
                exbuf.at[pl.ds(g * 16, 16)],
                den_sh.at[dstg[g]], semsc).wait()

    def start(i, b):
        drain_scatters(b)
        fire(i, b)

    def compute(i, b, prefetch=True):
        srcv, dstv, dstg, xlbuf, xrbuf, exbuf, semg, semsc, semi = bufs[b]
        pltpu.make_async_copy(xl_hbm.at[srcv], xlbuf, semg).wait()
        pltpu.make_async_copy(xr_hbm.at[dstv], xrbuf, semg).wait()
        for g in range(GPC):
            dstg[g][...] = dstv[pl.ds(g * 16, 16)]
        if prefetch:
            base2 = (wid * NCHUNK + i + 2) * C
            pltpu.async_copy(src_hbm.at[pl.ds(base2, C)], srcv, semi)
            pltpu.async_copy(dst_hbm.at[pl.ds(base2, C)], dstv, semi)
        for g in range(C // 16):
            @plsc.parallel_loop(0, 16, unroll=4)
            def _(k):
                e = g * 16 + k
                acc = jnp.zeros((16,), jnp.float32)
                for j in range(8):
                    v = xlbuf[e, pl.ds(16 * j, 16)] + xrbuf[e, pl.ds(16 * j, 16)]
                    v = v * jnp.where(v > 0.0, 1.0, 0.2)
                    acc = acc + v * att_regs[j]
                accbuf[pl.ds(k * 16, 16)] = acc
            logits = plsc.load_gather(accbuf, [row_iota * 16])
            for k in range(1, 16):
                logits = logits + plsc.load_gather(accbuf, [row_iota * 16 + k])
            exbuf[pl.ds(g * 16, 16)] = jnp.exp(logits)

            @plsc.parallel_loop(0, 16, unroll=4)
            def _(k):
                e = g * 16 + k
                exs = plsc.load_gather(exbuf, [jnp.full((16,), e, jnp.int32)])
                for j in range(8):
                    xlbuf[e, pl.ds(16 * j, 16)] = (
                        xlbuf[e, pl.ds(16 * j, 16)] * exs)
            pltpu.async_copy(xlbuf.at[pl.ds(g * 16, 16)],
                             num_sh.at[dstg[g]], semsc, add=True)
            pltpu.async_copy(exbuf.at[pl.ds(g * 16, 16)],
                             den_sh.at[dstg[g]], semsc, add=True)

    fire_first(0, 0)
    fire_first(1, 1)
    compute(0, 0)

    def pair_body(p, carry):
        c0 = 2 * p + 2
        start(c0, 0)
        compute(c0 - 1, 1)
        start(c0 + 1, 1)
        compute(c0, 0)
        return carry

    lax.fori_loop(0, (NCHUNK - 3) // 2, pair_body, 0)
    start(NCHUNK - 1, 0)
    compute(NCHUNK - 2, 1, prefetch=False)
    compute(NCHUNK - 1, 0, prefetch=False)
    drain_scatters(1)
    drain_scatters(0)
    plsc.subcore_barrier()

    pltpu.sync_copy(num_sh.at[pl.ds(sid * NPT, NPT)],
                    num_out.at[cid, pl.ds(sid * NPT, NPT)])

    @pl.when(sid == 0)
    def _():
        pltpu.sync_copy(den_sh, den_out.at[cid, 0])


_sc_edge = pl.kernel(
    _sc_edge_body,
    out_type=(
        jax.ShapeDtypeStruct((NC, NPAD, D), jnp.float32),
        jax.ShapeDtypeStruct((NC, 1, NPAD), jnp.float32),
    ),
    mesh=plsc.VectorSubcoreMesh(core_axis_name="c", subcore_axis_name="s"),
    compiler_params=pltpu.CompilerParams(needs_layout_passes=False),
    scratch_types=[
        pltpu.VMEM_SHARED((NPAD, D), jnp.float32),
        pltpu.VMEM_SHARED((NPAD,), jnp.float32),
        pltpu.VMEM((D,), jnp.float32),
        pltpu.VMEM((C,), jnp.int32),
        pltpu.VMEM((C,), jnp.int32),
        tuple(pltpu.VMEM((16,), jnp.int32) for _ in range(GPC)),
        pltpu.VMEM((C, D), jnp.float32),
        pltpu.VMEM((C, D), jnp.float32),
        pltpu.VMEM((C,), jnp.float32),
        pltpu.VMEM((C,), jnp.int32),
        pltpu.VMEM((C,), jnp.int32),
        tuple(pltpu.VMEM((16,), jnp.int32) for _ in range(GPC)),
        pltpu.VMEM((C, D), jnp.float32),
        pltpu.VMEM((C, D), jnp.float32),
        pltpu.VMEM((C,), jnp.float32),
        pltpu.VMEM((256,), jnp.float32),
        pltpu.SemaphoreType.DMA,
        pltpu.SemaphoreType.DMA,
        pltpu.SemaphoreType.DMA,
        pltpu.SemaphoreType.DMA,
        pltpu.SemaphoreType.DMA,
        pltpu.SemaphoreType.DMA,
    ],
)


def _proj_body(x_ref, wl_ref, wr_ref, xl_ref, xr_ref):
    h = x_ref[...]
    xl_ref[...] = jnp.dot(h, wl_ref[...], preferred_element_type=jnp.float32)
    xr_ref[...] = jnp.dot(h, wr_ref[...], preferred_element_type=jnp.float32)


def _tc_proj(x, wl, wr):
    return pl.pallas_call(
        _proj_body,
        grid=(GRID,),
        in_specs=[
            pl.BlockSpec((R, D), lambda i: (i, 0)),
            pl.BlockSpec((D, D), lambda i: (0, 0)),
            pl.BlockSpec((D, D), lambda i: (0, 0)),
        ],
        out_specs=[
            pl.BlockSpec((R, D), lambda i: (i, 0)),
            pl.BlockSpec((R, D), lambda i: (i, 0)),
        ],
        out_shape=[
            jax.ShapeDtypeStruct((N, D), jnp.float32),
            jax.ShapeDtypeStruct((N, D), jnp.float32),
        ],
    )(x, wl, wr)


def _combine_h(num_ref, den_ref):
    i = pl.program_id(0)
    den = den_ref[0, 0, pl.ds(i * RP, RP)] + den_ref[1, 0, pl.ds(i * RP, RP)]
    num = num_ref[0] + num_ref[1]
    den = den[:, None]
    h = jnp.where(den > 0.0, num / den, 0.0)
    return jnp.where(h > 0.0, h, jnp.exp(h) - 1.0)


def _mid_body(num_ref, den_ref, wl_ref, wr_ref, xl_ref, xr_ref):
    h = _combine_h(num_ref, den_ref)
    xl_ref[...] = jnp.dot(h, wl_ref[...], preferred_element_type=jnp.float32)
    xr_ref[...] = jnp.dot(h, wr_ref[...], preferred_element_type=jnp.float32)


def _tc_mid(num, den, wl, wr):
    return pl.pallas_call(
        _mid_body,
        grid=(GRIDP,),
        in_specs=[
            pl.BlockSpec((NC, RP, D), lambda i: (0, i, 0)),
            pl.BlockSpec((NC, 1, NPAD), lambda i: (0, 0, 0)),
            pl.BlockSpec((D, D), lambda i: (0, 0)),
            pl.BlockSpec((D, D), lambda i: (0, 0)),
        ],
        out_specs=[
            pl.BlockSpec((RP, D), lambda i: (i, 0)),
            pl.BlockSpec((RP, D), lambda i: (i, 0)),
        ],
        out_shape=[
            jax.ShapeDtypeStruct((NPAD, D), jnp.float32),
            jax.ShapeDtypeStruct((NPAD, D), jnp.float32),
        ],
    )(num, den, wl, wr)


def _final_body(num_ref, den_ref, out_ref):
    i = pl.program_id(0)
    den = den_ref[0, 0, pl.ds(i * RP, RP)] + den_ref[1, 0, pl.ds(i * RP, RP)]
    num = num_ref[0] + num_ref[1]
    den = den[:, None]
    h = jnp.where(den > 0.0, num / den, 0.0)
    out_ref[...] = jnp.where(h > 0.0, h, 128.0 * (jnp.exp(h) - 1.0))


def _tc_final(num, den):
    return pl.pallas_call(
        _final_body,
        grid=(GRIDP,),
        in_specs=[
            pl.BlockSpec((NC, RP, D), lambda i: (0, i, 0)),
            pl.BlockSpec((NC, 1, NPAD), lambda i: (0, 0, 0)),
        ],
        out_specs=pl.BlockSpec((RP, D), lambda i: (i, 0)),
        out_shape=jax.ShapeDtypeStruct((NPAD, D), jnp.float32),
    )(num, den)


def kernel(x, edge_index, Wl1, Wr1, att1, Wl2, Wr2, att2, Wl3, Wr3, att3,
           Wl4, Wr4, att4):
    src = edge_index[0]
    dst = edge_index[1]
    znd = jnp.zeros((NPAD, D), jnp.float32)
    zn = jnp.zeros((NPAD,), jnp.float32)

    xl, xr = _tc_proj(x, Wl1, Wr1)
    num, den = _sc_edge(xl, xr, att1, src, dst, znd, zn)
    xl, xr = _tc_mid(num, den, Wl2, Wr2)
    num, den = _sc_edge(xl, xr, att2, src, dst, znd, zn)
    xl, xr = _tc_mid(num, den, Wl3, Wr3)
    num, den = _sc_edge(xl, xr, att3, src, dst, znd, zn)
    xl, xr = _tc_mid(num, den, Wl4, Wr4)
    num, den = _sc_edge(xl, xr, att4, src, dst, znd, zn)
    return _tc_final(num, den)[:N]
